# Initial kernel scaffold; baseline (speedup 1.0000x reference)
import functools

import jax
import jax.numpy as jnp
from jax import lax
from jax.experimental import pallas as pl
from jax.experimental.pallas import tpu as pltpu
from jax.experimental.pallas import tpu_sc as plsc

N = 10000
F = 256
FH = 128
E = 160000
P = 8000
NB = 10
A = 1000
R = NB * P
LOW, HIGH = 0.0, 480.0

NC, NS = 2, 16
NW = NC * NS
CH = 128
EC = E // CH
RC = R // CH
ROWS_PER_TILE = N // NS

_mesh = plsc.VectorSubcoreMesh(core_axis_name="c", subcore_axis_name="s")


@functools.partial(
    pl.kernel, mesh=_mesh,
    out_type=jax.ShapeDtypeStruct((NC, N, 16), jnp.float32),
    scratch_types=[pltpu.VMEM((CH,), jnp.int32),
                   pltpu.VMEM((CH, 16), jnp.float32),
                   pltpu.VMEM_SHARED((N, 16), jnp.float32),
                   pltpu.SemaphoreType.DMA],
)
def _sc_deg(dst_hbm, ones_hbm, zeros_hbm, out_hbm, idx_v, ones_v, acc_sh, sem):
    c = lax.axis_index("c")
    s = lax.axis_index("s")
    wid = s * NC + c
    pltpu.sync_copy(ones_hbm, ones_v)
    sl = pl.ds(s * ROWS_PER_TILE, ROWS_PER_TILE)
    pltpu.sync_copy(zeros_hbm, acc_sh.at[sl])
    plsc.subcore_barrier()

    @pl.loop(wid, EC, step=NW)
    def _(j):
        pltpu.sync_copy(dst_hbm.at[pl.ds(j * CH, CH)], idx_v)
        pltpu.sync_copy(ones_v, acc_sh.at[idx_v], add=True)

    plsc.subcore_barrier()
    pltpu.sync_copy(acc_sh.at[sl], out_hbm.at[c].at[sl])


@functools.partial(
    pl.kernel, mesh=_mesh,
    out_type=jax.ShapeDtypeStruct((NC, N, FH), jnp.float32),
    scratch_types=[pltpu.VMEM((CH,), jnp.int32),
                   pltpu.VMEM((CH,), jnp.int32),
                   pltpu.VMEM((CH, FH), jnp.float32),
                   pltpu.VMEM_SHARED((N, FH), jnp.float32),
                   pltpu.SemaphoreType.DMA],
)
def _sc_gcn_agg(src_hbm, dst_hbm, xsp_hbm, zeros_hbm, out_hbm,
                sidx, didx, rbuf, acc_sh, sem):
    c = lax.axis_index("c")
    s = lax.axis_index("s")
    sl = pl.ds(s * ROWS_PER_TILE, ROWS_PER_TILE)
    pltpu.sync_copy(zeros_hbm, acc_sh.at[sl])
    plsc.subcore_barrier()

    @pl.loop(s, EC, step=NS)
    def _(j):
        pltpu.sync_copy(src_hbm.at[pl.ds(j * CH, CH)], sidx)
        pltpu.sync_copy(dst_hbm.at[pl.ds(j * CH, CH)], didx)
        pltpu.async_copy(xsp_hbm.at[c].at[sidx], rbuf, sem).wait()
        pltpu.sync_copy(rbuf, acc_sh.at[didx], add=True)

    plsc.subcore_barrier()
    pltpu.sync_copy(acc_sh.at[sl], out_hbm.at[c].at[sl])


@functools.partial(
    pl.kernel, mesh=_mesh,
    out_type=jax.ShapeDtypeStruct((2, R, F), jnp.float32),
    scratch_types=[pltpu.VMEM((CH,), jnp.int32),
                   pltpu.VMEM((CH,), jnp.int32),
                   pltpu.VMEM((CH, F), jnp.float32),
                   pltpu.VMEM((CH, F), jnp.float32),
                   pltpu.SemaphoreType.DMA,
                   pltpu.SemaphoreType.DMA],
)
def _sc_pair_gather(xs2_hbm, xd2_hbm, i0_hbm, i1_hbm, out_hbm,
                    idx0, idx1, rb0, rb1, sem0, sem1):
    c = lax.axis_index("c")
    s = lax.axis_index("s")
    wid = s * NC + c

    @pl.loop(wid, RC, step=NW)
    def _(j):
        rows = pl.ds(j * CH, CH)
        pltpu.sync_copy(i0_hbm.at[rows], idx0)
        pltpu.sync_copy(i1_hbm.at[rows], idx1)
        cp0 = pltpu.async_copy(xs2_hbm.at[idx0], rb0, sem0)
        cp1 = pltpu.async_copy(xd2_hbm.at[idx1], rb1, sem1)
        cp0.wait()
        cp1.wait()
        pltpu.sync_copy(rb0, out_hbm.at[0].at[rows])
        pltpu.sync_copy(rb1, out_hbm.at[1].at[rows])


def _tc_scale_split(state, Wg, degp):
    BLK = 1000

    def body(st_ref, wg_ref, dg_ref, out_ref):
        xw = jnp.dot(st_ref[...], wg_ref[...], preferred_element_type=jnp.float32)
        deg = dg_ref[0] + dg_ref[1] + 1.0
        dinv = lax.rsqrt(deg[:, 0:1])
        xs = xw * dinv
        out_ref[0] = xs[:, :FH]
        out_ref[1] = xs[:, FH:]

    return pl.pallas_call(
        body,
        grid=(N // BLK,),
        in_specs=[pl.BlockSpec((BLK, F), lambda i: (i, 0)),
                  pl.BlockSpec((F, F), lambda i: (0, 0)),
                  pl.BlockSpec((NC, BLK, 16), lambda i: (0, i, 0))],
        out_specs=pl.BlockSpec((NC, BLK, FH), lambda i: (0, i, 0)),
        out_shape=jax.ShapeDtypeStruct((NC, N, FH), jnp.float32),
    )(state, Wg, degp)


def _tc_node_mlp_in(accp, xsp, degp, state, bg, W1t, W1b):
    BLK = 1000

    def body(ac_ref, xs_ref, dg_ref, st_ref, bg_ref, w1t_ref, w1b_ref,
             o1_ref, o2_ref):
        acc = jnp.concatenate([ac_ref[0], ac_ref[1]], axis=1)
        xs = jnp.concatenate([xs_ref[0], xs_ref[1]], axis=1)
        deg = dg_ref[0] + dg_ref[1] + 1.0
        dinv = lax.rsqrt(deg[:, 0:1])
        gcn = (acc + xs) * dinv + bg_ref[...]
        x = jnp.maximum(gcn, 0.0) + st_ref[...]
        o1_ref[...] = jnp.dot(x, w1t_ref[...], preferred_element_type=jnp.float32)
        o2_ref[...] = jnp.dot(x, w1b_ref[...], preferred_element_type=jnp.float32)

    return pl.pallas_call(
        body,
        grid=(N // BLK,),
        in_specs=[pl.BlockSpec((NC, BLK, FH), lambda i: (0, i, 0)),
                  pl.BlockSpec((NC, BLK, FH), lambda i: (0, i, 0)),
                  pl.BlockSpec((NC, BLK, 16), lambda i: (0, i, 0)),
                  pl.BlockSpec((BLK, F), lambda i: (i, 0)),
                  pl.BlockSpec((F,), lambda i: (0,)),
                  pl.BlockSpec((F, F), lambda i: (0, 0)),
                  pl.BlockSpec((F, F), lambda i: (0, 0))],
        out_specs=[pl.BlockSpec((BLK, F), lambda i: (i, 0)),
                   pl.BlockSpec((BLK, F), lambda i: (i, 0))],
        out_shape=[jax.ShapeDtypeStruct((N, F), jnp.float32),
                   jax.ShapeDtypeStruct((N, F), jnp.float32)],
    )(accp, xsp, degp, state, bg, W1t, W1b)


def _tc_head(g2, b1, W2, b2, Wmu, bmu):
    BLK = 3200

    def body(g_ref, b1_ref, w2_ref, b2_ref, wmu_ref, bmu_ref, o_ref):
        h = g_ref[0] + g_ref[1] + b1_ref[...]
        h = jnp.where(h > 0, h, 0.01 * h)
        h2 = jnp.dot(h, w2_ref[...], preferred_element_type=jnp.float32) + b2_ref[...]
        h2 = jnp.where(h2 > 0, h2, 0.01 * h2)
        m = jnp.dot(h2, wmu_ref[...], preferred_element_type=jnp.float32) + bmu_ref[...]
        mu = jax.nn.softplus(m)
        act = (jnp.tanh(mu) + 1.0) * (0.5 * (HIGH - LOW)) + LOW
        o_ref[...] = jnp.clip(act, LOW, HIGH)

    return pl.pallas_call(
        body,
        grid=(R // BLK,),
        in_specs=[pl.BlockSpec((2, BLK, F), lambda i: (0, i, 0)),
                  pl.BlockSpec((F,), lambda i: (0,)),
                  pl.BlockSpec((F, F), lambda i: (0, 0)),
                  pl.BlockSpec((F,), lambda i: (0,)),
                  pl.BlockSpec((F, 1), lambda i: (0, 0)),
                  pl.BlockSpec((1,), lambda i: (0,))],
        out_specs=pl.BlockSpec((BLK, 1), lambda i: (i, 0)),
        out_shape=jax.ShapeDtypeStruct((R, 1), jnp.float32),
    )(g2, b1, W2, b2, Wmu, bmu)


def kernel(state, edge_index, edges, deterministic,
           Wg, bg, W1, b1, W2, b2, Wmu, bmu, Wsig, bsig):
    del deterministic, Wsig, bsig
    src = edge_index[0]
    dst = edge_index[1]
    ones16 = jnp.ones((CH, 16), jnp.float32)
    zeros16 = jnp.zeros((ROWS_PER_TILE, 16), jnp.float32)
    zerosH = jnp.zeros((ROWS_PER_TILE, FH), jnp.float32)

    degp = _sc_deg(dst, ones16, zeros16)
    xsp = _tc_scale_split(state, Wg, degp)
    accp = _sc_gcn_agg(src, dst, xsp, zerosH)
    xs2, xd2 = _tc_node_mlp_in(accp, xsp, degp, state, bg,
                               W1[:F, :], W1[F:, :])

    boff = (jnp.arange(NB, dtype=jnp.int32) * A)[:, None]
    i0 = (boff + edges[:, 0][None, :]).reshape(-1)
    i1 = (boff + edges[:, 1][None, :]).reshape(-1)
    g2 = _sc_pair_gather(xs2, xd2, i0, i1)

    act = _tc_head(g2, b1, W2, b2, Wmu, bmu)
    return act.reshape(NB, P)


# speedup vs baseline: 7.3595x; 7.3595x over previous
import functools

import jax
import jax.numpy as jnp
from jax import lax
from jax.experimental import pallas as pl
from jax.experimental.pallas import tpu as pltpu
from jax.experimental.pallas import tpu_sc as plsc

N = 10000
NP = 10240
F = 256
FH = 128
E = 160000
P = 8000
NB = 10
A = 1000
R = NB * P
LOW, HIGH = 0.0, 480.0

NC, NS = 2, 16
NW = NC * NS
CH = 128
EC = E // CH
RC = R // CH
ROWS_PER_TILE = NP // NS

_mesh = plsc.VectorSubcoreMesh(core_axis_name="c", subcore_axis_name="s")


@functools.partial(
    pl.kernel, mesh=_mesh,
    out_type=jax.ShapeDtypeStruct((NC, NP, FH), jnp.float32),
    scratch_types=[pltpu.VMEM((CH,), jnp.int32),
                   pltpu.VMEM((CH, FH), jnp.float32),
                   pltpu.VMEM_SHARED((NP, FH), jnp.float32),
                   pltpu.SemaphoreType.DMA],
)
def _sc_deg(dst_hbm, ones_hbm, zeros_hbm, out_hbm, idx_v, ones_v, acc_sh, sem):
    c = lax.axis_index("c")
    s = lax.axis_index("s")
    wid = s * NC + c
    pltpu.sync_copy(ones_hbm, ones_v)
    sl = pl.ds(s * ROWS_PER_TILE, ROWS_PER_TILE)
    pltpu.sync_copy(zeros_hbm, acc_sh.at[sl])
    plsc.subcore_barrier()

    @pl.loop(wid, EC, step=NW)
    def _(j):
        pltpu.sync_copy(dst_hbm.at[pl.ds(j * CH, CH)], idx_v)
        pltpu.sync_copy(ones_v, acc_sh.at[idx_v], add=True)

    plsc.subcore_barrier()
    pltpu.sync_copy(acc_sh.at[sl], out_hbm.at[c].at[sl])


@functools.partial(
    pl.kernel, mesh=_mesh,
    out_type=jax.ShapeDtypeStruct((NC, NP, FH), jnp.float32),
    scratch_types=[pltpu.VMEM((CH,), jnp.int32),
                   pltpu.VMEM((CH,), jnp.int32),
                   pltpu.VMEM((CH, FH), jnp.float32),
                   pltpu.VMEM_SHARED((NP, FH), jnp.float32),
                   pltpu.SemaphoreType.DMA],
)
def _sc_gcn_agg(src_hbm, dst_hbm, xsp_hbm, zeros_hbm, out_hbm,
                sidx, didx, rbuf, acc_sh, sem):
    c = lax.axis_index("c")
    s = lax.axis_index("s")
    sl = pl.ds(s * ROWS_PER_TILE, ROWS_PER_TILE)
    pltpu.sync_copy(zeros_hbm, acc_sh.at[sl])
    plsc.subcore_barrier()

    @pl.loop(s, EC, step=NS)
    def _(j):
        pltpu.sync_copy(src_hbm.at[pl.ds(j * CH, CH)], sidx)
        pltpu.sync_copy(dst_hbm.at[pl.ds(j * CH, CH)], didx)
        pltpu.async_copy(xsp_hbm.at[c].at[sidx], rbuf, sem).wait()
        pltpu.sync_copy(rbuf, acc_sh.at[didx], add=True)

    plsc.subcore_barrier()
    pltpu.sync_copy(acc_sh.at[sl], out_hbm.at[c].at[sl])


@functools.partial(
    pl.kernel, mesh=_mesh,
    out_type=jax.ShapeDtypeStruct((2, R, F), jnp.float32),
    scratch_types=[pltpu.VMEM((CH,), jnp.int32),
                   pltpu.VMEM((CH,), jnp.int32),
                   pltpu.VMEM((CH, F), jnp.float32),
                   pltpu.VMEM((CH, F), jnp.float32),
                   pltpu.SemaphoreType.DMA,
                   pltpu.SemaphoreType.DMA],
)
def _sc_pair_gather(xs2_hbm, xd2_hbm, i0_hbm, i1_hbm, out_hbm,
                    idx0, idx1, rb0, rb1, sem0, sem1):
    c = lax.axis_index("c")
    s = lax.axis_index("s")
    wid = s * NC + c

    @pl.loop(wid, RC, step=NW)
    def _(j):
        rows = pl.ds(j * CH, CH)
        pltpu.sync_copy(i0_hbm.at[rows], idx0)
        pltpu.sync_copy(i1_hbm.at[rows], idx1)
        cp0 = pltpu.async_copy(xs2_hbm.at[idx0], rb0, sem0)
        cp1 = pltpu.async_copy(xd2_hbm.at[idx1], rb1, sem1)
        cp0.wait()
        cp1.wait()
        pltpu.sync_copy(rb0, out_hbm.at[0].at[rows])
        pltpu.sync_copy(rb1, out_hbm.at[1].at[rows])


def _tc_scale_split(state, Wg, degp):
    BLK = 1000

    def body(st_ref, wg_ref, dg_ref, out_ref):
        xw = jnp.dot(st_ref[...], wg_ref[...], preferred_element_type=jnp.float32)
        deg = dg_ref[0] + dg_ref[1] + 1.0
        dinv = lax.rsqrt(deg[:, 0:1])
        xs = xw * dinv
        out_ref[0] = xs[:, :FH]
        out_ref[1] = xs[:, FH:]

    return pl.pallas_call(
        body,
        grid=(N // BLK,),
        in_specs=[pl.BlockSpec((BLK, F), lambda i: (i, 0)),
                  pl.BlockSpec((F, F), lambda i: (0, 0)),
                  pl.BlockSpec((NC, BLK, FH), lambda i: (0, i, 0))],
        out_specs=pl.BlockSpec((NC, BLK, FH), lambda i: (0, i, 0)),
        out_shape=jax.ShapeDtypeStruct((NC, NP, FH), jnp.float32),
    )(state, Wg, degp)


def _tc_node_mlp_in(accp, xsp, degp, state, bg, W1t, W1b):
    BLK = 1000

    def body(ac_ref, xs_ref, dg_ref, st_ref, bg_ref, w1t_ref, w1b_ref,
             o1_ref, o2_ref):
        acc = jnp.concatenate([ac_ref[0], ac_ref[1]], axis=1)
        xs = jnp.concatenate([xs_ref[0], xs_ref[1]], axis=1)
        deg = dg_ref[0] + dg_ref[1] + 1.0
        dinv = lax.rsqrt(deg[:, 0:1])
        gcn = (acc + xs) * dinv + bg_ref[...]
        x = jnp.maximum(gcn, 0.0) + st_ref[...]
        o1_ref[...] = jnp.dot(x, w1t_ref[...], preferred_element_type=jnp.float32)
        o2_ref[...] = jnp.dot(x, w1b_ref[...], preferred_element_type=jnp.float32)

    return pl.pallas_call(
        body,
        grid=(N // BLK,),
        in_specs=[pl.BlockSpec((NC, BLK, FH), lambda i: (0, i, 0)),
                  pl.BlockSpec((NC, BLK, FH), lambda i: (0, i, 0)),
                  pl.BlockSpec((NC, BLK, FH), lambda i: (0, i, 0)),
                  pl.BlockSpec((BLK, F), lambda i: (i, 0)),
                  pl.BlockSpec((F,), lambda i: (0,)),
                  pl.BlockSpec((F, F), lambda i: (0, 0)),
                  pl.BlockSpec((F, F), lambda i: (0, 0))],
        out_specs=[pl.BlockSpec((BLK, F), lambda i: (i, 0)),
                   pl.BlockSpec((BLK, F), lambda i: (i, 0))],
        out_shape=[jax.ShapeDtypeStruct((N, F), jnp.float32),
                   jax.ShapeDtypeStruct((N, F), jnp.float32)],
    )(accp, xsp, degp, state, bg, W1t, W1b)


def _tc_head(g2, b1, W2, b2, Wmu, bmu):
    BLK = 3200

    def body(g_ref, b1_ref, w2_ref, b2_ref, wmu_ref, bmu_ref, o_ref):
        h = g_ref[0] + g_ref[1] + b1_ref[...]
        h = jnp.where(h > 0, h, 0.01 * h)
        h2 = jnp.dot(h, w2_ref[...], preferred_element_type=jnp.float32) + b2_ref[...]
        h2 = jnp.where(h2 > 0, h2, 0.01 * h2)
        m = jnp.dot(h2, wmu_ref[...], preferred_element_type=jnp.float32) + bmu_ref[...]
        mu = jax.nn.softplus(m)
        act = (jnp.tanh(mu) + 1.0) * (0.5 * (HIGH - LOW)) + LOW
        o_ref[...] = jnp.clip(act, LOW, HIGH)

    return pl.pallas_call(
        body,
        grid=(R // BLK,),
        in_specs=[pl.BlockSpec((2, BLK, F), lambda i: (0, i, 0)),
                  pl.BlockSpec((F,), lambda i: (0,)),
                  pl.BlockSpec((F, F), lambda i: (0, 0)),
                  pl.BlockSpec((F,), lambda i: (0,)),
                  pl.BlockSpec((F, 1), lambda i: (0, 0)),
                  pl.BlockSpec((1,), lambda i: (0,))],
        out_specs=pl.BlockSpec((BLK, 1), lambda i: (i, 0)),
        out_shape=jax.ShapeDtypeStruct((R, 1), jnp.float32),
    )(g2, b1, W2, b2, Wmu, bmu)


def kernel(state, edge_index, edges, deterministic,
           Wg, bg, W1, b1, W2, b2, Wmu, bmu, Wsig, bsig):
    del deterministic, Wsig, bsig
    src = edge_index[0]
    dst = edge_index[1]
    onesH = jnp.ones((CH, FH), jnp.float32)
    zerosH = jnp.zeros((ROWS_PER_TILE, FH), jnp.float32)

    degp = _sc_deg(dst, onesH, zerosH)
    xsp = _tc_scale_split(state, Wg, degp)
    accp = _sc_gcn_agg(src, dst, xsp, zerosH)
    xs2, xd2 = _tc_node_mlp_in(accp, xsp, degp, state, bg,
                               W1[:F, :], W1[F:, :])

    boff = (jnp.arange(NB, dtype=jnp.int32) * A)[:, None]
    i0 = (boff + edges[:, 0][None, :]).reshape(-1)
    i1 = (boff + edges[:, 1][None, :]).reshape(-1)
    g2 = _sc_pair_gather(xs2, xd2, i0, i1)

    act = _tc_head(g2, b1, W2, b2, Wmu, bmu)
    return act.reshape(NB, P)
